# baseline (speedup 1.0000x reference)
import jax
import jax.numpy as jnp
from jax import lax
from jax.experimental import pallas as pl
from jax.experimental.pallas import tpu as pltpu
from jax.experimental.pallas import tpu_sc as plsc

B, L = 16384, 50
N = B * L
ASN_D = 19
GEO_V, GEO_D = 252, 8
OUT_D = 32

NC, NS = 2, 16
NW = NC * NS
PER_W = N // NW
T = 1024
NCHUNK = PER_W // T
G = T // 16
IDX_ROWS = T // 128
NBUF = 3


def _body_fixed(asn_table, geo_table, ips_table, in3d, asn_idx, out,
                inb0, inb1, inb2, aidx0, aidx1, aidx2,
                outb0, outb1, outb2, geo_tab_v, ips_tab_v,
                gsem0, gsem1, gsem2, osem0, osem1, osem2):
    pltpu.sync_copy(geo_table, geo_tab_v)
    pltpu.sync_copy(ips_table, ips_tab_v)
    _body_inner(asn_table, in3d, asn_idx, out,
                [inb0, inb1, inb2], [aidx0, aidx1, aidx2],
                [outb0, outb1, outb2], geo_tab_v, ips_tab_v,
                [gsem0, gsem1, gsem2], [osem0, osem1, osem2])


def _body_inner(asn_table, in3d, asn_idx, out, inb, aidx, outb,
                geo_tab_v, ips_tab_v, gsem, osem):
    wid = lax.axis_index("s") * NC + lax.axis_index("c")
    wc0 = wid * NCHUNK
    iota = lax.iota(jnp.int32, 16)

    gather_descs = [None] * NCHUNK
    out_descs = [None] * NCHUNK

    def stage_and_fire(j):
        b = j % NBUF
        c = wc0 + j
        pltpu.sync_copy(in3d.at[c], inb[b])
        pltpu.sync_copy(asn_idx.at[pl.ds(pl.multiple_of(c * IDX_ROWS, 8),
                                         IDX_ROWS)], aidx[b])
        descs = []
        for r in range(IDX_ROWS):
            descs.append(pltpu.async_copy(
                asn_table.at[aidx[b].at[r]],
                outb[b].at[pl.ds(r * 128, 128)],
                gsem[b]))
        gather_descs[j] = descs

    def assemble(j):
        b = j % NBUF

        def group_body(g, carry):
            t0 = g * 16
            rows = iota + t0
            latv = plsc.bitcast(inb[b][2, pl.ds(t0, 16)], jnp.float32)
            lonv = plsc.bitcast(inb[b][3, pl.ds(t0, 16)], jnp.float32)
            plsc.store_scatter(outb[b], [rows, jnp.zeros((16,), jnp.int32)],
                               latv)
            plsc.store_scatter(outb[b], [rows, jnp.ones((16,), jnp.int32)],
                               lonv)
            gi = inb[b][0, pl.ds(t0, 16)]
            for col in range(GEO_D):
                v = plsc.load_gather(geo_tab_v,
                                     [gi, jnp.full((16,), col, jnp.int32)])
                plsc.store_scatter(outb[b],
                                   [rows, jnp.full((16,), 21 + col,
                                                   jnp.int32)], v)
            pi = inb[b][1, pl.ds(t0, 16)]
            for col in range(3):
                v = plsc.load_gather(ips_tab_v,
                                     [pi, jnp.full((16,), col, jnp.int32)])
                plsc.store_scatter(outb[b],
                                   [rows, jnp.full((16,), 29 + col,
                                                   jnp.int32)], v)
            return carry

        lax.fori_loop(0, G, group_body, 0)

    stage_and_fire(0)
    stage_and_fire(1)
    for j in range(NCHUNK):
        b = j % NBUF
        if j + 2 < NCHUNK:
            if j + 2 >= NBUF:
                out_descs[j - 1].wait()
            stage_and_fire(j + 2)
        for d in gather_descs[j]:
            d.wait()
        assemble(j)
        base = (wc0 + j) * T
        out_descs[j] = pltpu.async_copy(
            outb[b], out.at[pl.ds(base, T)], osem[b])
    for j in range(NCHUNK - 3, NCHUNK):
        out_descs[j].wait()


@jax.jit
def _run(asn_table, geo_table, ips_table, in3d, asn_idx):
    mesh = plsc.VectorSubcoreMesh(core_axis_name="c", subcore_axis_name="s")
    return pl.kernel(
        _body_fixed,
        out_type=jax.ShapeDtypeStruct((N, OUT_D), jnp.float32),
        mesh=mesh,
        compiler_params=pltpu.CompilerParams(
            needs_layout_passes=False, use_tc_tiling_on_sc=False),
        scratch_types=(
            [pltpu.VMEM((4, T), jnp.int32) for _ in range(NBUF)]
            + [pltpu.VMEM((IDX_ROWS, 128), jnp.int32) for _ in range(NBUF)]
            + [pltpu.VMEM((T, OUT_D), jnp.float32) for _ in range(NBUF)]
            + [pltpu.VMEM((GEO_V, GEO_D), jnp.float32),
               pltpu.VMEM((8, 4), jnp.float32)]
            + [pltpu.SemaphoreType.DMA for _ in range(2 * NBUF)]
        ),
    )(asn_table, geo_table, ips_table, in3d, asn_idx)


def kernel(x_lat, x_long, x_asn, x_geo_cc, x_ip_source,
           asn_table, geo_cc_table, ip_source_table):
    asn_idx = x_asn.reshape(N // 128, 128).astype(jnp.int32)
    geo_i = x_geo_cc.reshape(N).astype(jnp.int32)
    ips_i = x_ip_source.reshape(N).astype(jnp.int32)
    lat_b = lax.bitcast_convert_type(x_lat.reshape(N), jnp.int32)
    lon_b = lax.bitcast_convert_type(x_long.reshape(N), jnp.int32)
    in3d = (jnp.stack([geo_i, ips_i, lat_b, lon_b], axis=0)
            .reshape(4, N // T, T).transpose(1, 0, 2))
    ips_pad = jnp.pad(ip_source_table, ((0, 3), (0, 1)))
    asn_pad = jnp.pad(asn_table, ((0, 0), (2, OUT_D - 2 - ASN_D)))
    out = _run(asn_pad, geo_cc_table, ips_pad, in3d, asn_idx)
    return out.reshape(B, L, OUT_D)
